# Initial kernel scaffold; baseline (speedup 1.0000x reference)
import functools

import jax
import jax.numpy as jnp
from jax import lax
from jax.experimental import pallas as pl
from jax.experimental.pallas import tpu as pltpu
from jax.experimental.pallas import tpu_sc as plsc

N = 10000
E = 320000
D = 128
NC = 2
NS = 16
CHUNK = 128
NCH = 80
EPT = NCH * CHUNK
E_PAD = NC * NS * EPT
NP = N + 16
STRIPE = NP // NS
DEGW = 16

_MESH = plsc.VectorSubcoreMesh(core_axis_name="c", subcore_axis_name="s")


def _spmm_body(with_deg, *refs):
    if with_deg:
        (feat, colsr, rowsr, zf, zd, p0, p1, d0, d1,
         cols_v, rows_v, gb0, gb1, ones_v, acc, dacc, sem0, sem1) = refs
    else:
        (feat, colsr, rowsr, zf, p0, p1,
         cols_v, rows_v, gb0, gb1, acc, sem0, sem1) = refs

    cid = lax.axis_index("c")
    sid = lax.axis_index("s")
    base = sid * STRIPE

    pltpu.sync_copy(colsr.at[cid, sid], cols_v)
    pltpu.sync_copy(rowsr.at[cid, sid], rows_v)
    pltpu.sync_copy(zf, acc.at[pl.ds(base, STRIPE)])
    if with_deg:
        pltpu.sync_copy(zd, dacc.at[pl.ds(base, STRIPE)])
        one = jnp.ones((16,), jnp.float32)
        for i in range(CHUNK):
            ones_v[i] = one
    plsc.subcore_barrier()

    pltpu.async_copy(feat.at[cols_v.at[0]], gb0, sem0)

    def step(i, carry):
        j = i * 2
        for b in range(2):
            gbc, semc = (gb0, sem0) if b == 0 else (gb1, sem1)
            gbn, semn = (gb1, sem1) if b == 0 else (gb0, sem0)
            jj = j + b
            nxt = jj + 1

            @pl.when(nxt < NCH)
            def _():
                pltpu.async_copy(feat.at[cols_v.at[nxt]], gbn, semn)

            pltpu.make_async_copy(feat.at[cols_v.at[jj]], gbc, semc).wait()
            pltpu.sync_copy(gbc, acc.at[rows_v.at[jj]], add=True)
            if with_deg:
                pltpu.sync_copy(ones_v, dacc.at[rows_v.at[jj]], add=True)
        return carry

    lax.fori_loop(0, NCH // 2, step, 0)

    plsc.subcore_barrier()

    @pl.when(cid == 0)
    def _():
        pltpu.sync_copy(acc.at[pl.ds(base, STRIPE)], p0.at[pl.ds(base, STRIPE)])
        if with_deg:
            pltpu.sync_copy(dacc.at[pl.ds(base, STRIPE)],
                            d0.at[pl.ds(base, STRIPE)])

    @pl.when(cid == 1)
    def _():
        pltpu.sync_copy(acc.at[pl.ds(base, STRIPE)], p1.at[pl.ds(base, STRIPE)])
        if with_deg:
            pltpu.sync_copy(dacc.at[pl.ds(base, STRIPE)],
                            d1.at[pl.ds(base, STRIPE)])


def _make_spmm(with_deg):
    outs = [jax.ShapeDtypeStruct((NP, D), jnp.float32)] * 2
    scratch = [
        pltpu.VMEM((NCH, CHUNK), jnp.int32),
        pltpu.VMEM((NCH, CHUNK), jnp.int32),
        pltpu.VMEM((CHUNK, D), jnp.float32),
        pltpu.VMEM((CHUNK, D), jnp.float32),
    ]
    if with_deg:
        outs += [jax.ShapeDtypeStruct((NP, DEGW), jnp.float32)] * 2
        scratch.append(pltpu.VMEM((CHUNK, DEGW), jnp.float32))
    scratch.append(pltpu.VMEM_SHARED((NP, D), jnp.float32))
    if with_deg:
        scratch.append(pltpu.VMEM_SHARED((NP, DEGW), jnp.float32))
    scratch += [pltpu.SemaphoreType.DMA, pltpu.SemaphoreType.DMA]
    return pl.kernel(
        functools.partial(_spmm_body, with_deg),
        out_type=tuple(outs),
        mesh=_MESH,
        scratch_types=scratch,
    )


_spmm_deg = _make_spmm(True)
_spmm = _make_spmm(False)

_RB = 1000


def _mm_body(x_ref, w_ref, o_ref):
    o_ref[...] = jnp.dot(x_ref[...], w_ref[...],
                         preferred_element_type=jnp.float32)


def _mm(x, w):
    return pl.pallas_call(
        _mm_body,
        grid=(N // _RB,),
        in_specs=[pl.BlockSpec((_RB, D), lambda i: (i, 0)),
                  pl.BlockSpec((D, D), lambda i: (0, 0))],
        out_specs=pl.BlockSpec((_RB, D), lambda i: (i, 0)),
        out_shape=jax.ShapeDtypeStruct((N, D), jnp.float32),
    )(x, w)


def _layer2_body(p0, p1, d0, d1, b1, w2, o):
    deg = d0[:, 0:1] + d1[:, 0:1]
    rinv = jnp.where(deg > 0, 1.0 / deg, 0.0)
    h = (p0[...] + p1[...]) * rinv + b1[...]
    h = jnp.where(h >= 0, h, 0.2 * h)
    o[...] = jnp.dot(h, w2[...], preferred_element_type=jnp.float32)


def _layer2(p0, p1, d0, d1, b1, w2):
    return pl.pallas_call(
        _layer2_body,
        grid=(N // _RB,),
        in_specs=[pl.BlockSpec((_RB, D), lambda i: (i, 0)),
                  pl.BlockSpec((_RB, D), lambda i: (i, 0)),
                  pl.BlockSpec((_RB, DEGW), lambda i: (i, 0)),
                  pl.BlockSpec((_RB, DEGW), lambda i: (i, 0)),
                  pl.BlockSpec((1, D), lambda i: (0, 0)),
                  pl.BlockSpec((D, D), lambda i: (0, 0))],
        out_specs=pl.BlockSpec((_RB, D), lambda i: (i, 0)),
        out_shape=jax.ShapeDtypeStruct((N, D), jnp.float32),
    )(p0, p1, d0, d1, b1, w2)


def _final_body(p0, p1, d0, d1, b2, o):
    deg = d0[:, 0:1] + d1[:, 0:1]
    rinv = jnp.where(deg > 0, 1.0 / deg, 0.0)
    h = (p0[...] + p1[...]) * rinv + b2[...]
    nrm = jnp.sqrt(jnp.sum(h * h, axis=1, keepdims=True))
    o[...] = h / jnp.maximum(nrm, 1e-12)


def _final(p0, p1, d0, d1, b2):
    return pl.pallas_call(
        _final_body,
        grid=(N // _RB,),
        in_specs=[pl.BlockSpec((_RB, D), lambda i: (i, 0)),
                  pl.BlockSpec((_RB, D), lambda i: (i, 0)),
                  pl.BlockSpec((_RB, DEGW), lambda i: (i, 0)),
                  pl.BlockSpec((_RB, DEGW), lambda i: (i, 0)),
                  pl.BlockSpec((1, D), lambda i: (0, 0))],
        out_specs=pl.BlockSpec((_RB, D), lambda i: (i, 0)),
        out_shape=jax.ShapeDtypeStruct((N, D), jnp.float32),
    )(p0, p1, d0, d1, b2)


def kernel(x, edge_index, W1, b1, W2, b2):
    cols = edge_index[:, 0]
    rows = edge_index[:, 1]
    pad = E_PAD - E
    cols = jnp.concatenate([cols, jnp.zeros((pad,), jnp.int32)])
    rows = jnp.concatenate([rows, jnp.full((pad,), N, jnp.int32)])
    colsr = cols.reshape(NC, NS, NCH, CHUNK)
    rowsr = rows.reshape(NC, NS, NCH, CHUNK)
    zf = jnp.zeros((STRIPE, D), jnp.float32)
    zd = jnp.zeros((STRIPE, DEGW), jnp.float32)

    y1 = _mm(x, W1)
    p0, p1, d0, d1 = _spmm_deg(y1, colsr, rowsr, zf, zd)
    y2 = _layer2(p0, p1, d0, d1, b1.reshape(1, D), W2)
    q0, q1 = _spmm(y2, colsr, rowsr, zf)
    return _final(q0, q1, d0, d1, b2.reshape(1, D))


# speedup vs baseline: 5.2372x; 5.2372x over previous
import jax
import jax.numpy as jnp
from jax import lax
from jax.experimental import pallas as pl
from jax.experimental.pallas import tpu as pltpu
from jax.experimental.pallas import tpu_sc as plsc

N = 10000
E = 320000
D = 128
NC = 2
NS = 16
CHUNK = 128
NCH = 80
IB = 16
NGRP = NCH // IB
EPT = NCH * CHUNK
E_PAD = NC * NS * EPT
PADTOT = float(E_PAD - E)
STRIPE = 632
LAST = N - 15 * STRIPE
DEGW = 128

_MESH = plsc.VectorSubcoreMesh(core_axis_name="c", subcore_axis_name="s",
                               num_cores=NC, num_subcores=NS)


def _stripe_copy(src, dst, sid):
    @pl.when(sid < 15)
    def _():
        pltpu.sync_copy(src.at[pl.ds(sid * STRIPE, STRIPE)],
                        dst.at[pl.ds(sid * STRIPE, STRIPE)])

    @pl.when(sid == 15)
    def _():
        pltpu.sync_copy(src.at[pl.ds(15 * STRIPE, LAST)],
                        dst.at[pl.ds(15 * STRIPE, LAST)])


def _zero_stripe(z, dst, sid):
    @pl.when(sid < 15)
    def _():
        pltpu.sync_copy(z, dst.at[pl.ds(sid * STRIPE, STRIPE)])

    @pl.when(sid == 15)
    def _():
        pltpu.sync_copy(z.at[pl.ds(0, LAST)], dst.at[pl.ds(15 * STRIPE, LAST)])


def _deg_body(rowsr, zd, d0, d1, rows_v, ones_v, dacc):
    cid = lax.axis_index("c")
    sid = lax.axis_index("s")

    pltpu.sync_copy(rowsr.at[cid, sid], rows_v)
    _zero_stripe(zd, dacc, sid)
    one = jnp.ones((16,), jnp.float32)
    for i in range(CHUNK):
        for l in range(DEGW // 16):
            ones_v[i, pl.ds(l * 16, 16)] = one
    plsc.subcore_barrier()

    def step(j, carry):
        pltpu.sync_copy(ones_v, dacc.at[rows_v.at[j]], add=True)
        return carry

    lax.fori_loop(0, NCH, step, 0)
    plsc.subcore_barrier()

    @pl.when(cid == 0)
    def _():
        _stripe_copy(dacc, d0, sid)

    @pl.when(cid == 1)
    def _():
        _stripe_copy(dacc, d1, sid)


_deg = pl.kernel(
    _deg_body,
    out_type=(jax.ShapeDtypeStruct((N, DEGW), jnp.float32),) * 2,
    mesh=_MESH,
    scratch_types=[
        pltpu.VMEM((NCH, CHUNK), jnp.int32),
        pltpu.VMEM((CHUNK, DEGW), jnp.float32),
        pltpu.VMEM_SHARED((N, DEGW), jnp.float32),
    ],
)


def _spmm_body(feat, colsr, rowsr, zf, p0, p1,
               cols_v, rows_v, gb0, gb1, acc, sem0, sem1):
    cid = lax.axis_index("c")
    sid = lax.axis_index("s")

    _zero_stripe(zf, acc, sid)
    plsc.subcore_barrier()

    def group(g, carry):
        pltpu.sync_copy(colsr.at[cid, sid, pl.ds(g * IB, IB)], cols_v)
        pltpu.sync_copy(rowsr.at[cid, sid, pl.ds(g * IB, IB)], rows_v)
        pltpu.async_copy(feat.at[cols_v.at[0]], gb0, sem0)

        def step(k, c2):
            for b in range(2):
                gbc, semc = (gb0, sem0) if b == 0 else (gb1, sem1)
                gbn, semn = (gb1, sem1) if b == 0 else (gb0, sem0)
                kk = 2 * k + b
                nxt = kk + 1

                @pl.when(nxt < IB)
                def _():
                    pltpu.async_copy(feat.at[cols_v.at[nxt]], gbn, semn)

                pltpu.make_async_copy(feat.at[cols_v.at[kk]], gbc, semc).wait()
                pltpu.sync_copy(gbc, acc.at[rows_v.at[kk]], add=True)
            return c2

        lax.fori_loop(0, IB // 2, step, 0)
        return carry

    lax.fori_loop(0, NGRP, group, 0)

    plsc.subcore_barrier()

    @pl.when(cid == 0)
    def _():
        _stripe_copy(acc, p0, sid)

    @pl.when(cid == 1)
    def _():
        _stripe_copy(acc, p1, sid)


_spmm = pl.kernel(
    _spmm_body,
    out_type=(jax.ShapeDtypeStruct((N, D), jnp.float32),) * 2,
    mesh=_MESH,
    scratch_types=[
        pltpu.VMEM((IB, CHUNK), jnp.int32),
        pltpu.VMEM((IB, CHUNK), jnp.int32),
        pltpu.VMEM((CHUNK, D), jnp.float32),
        pltpu.VMEM((CHUNK, D), jnp.float32),
        pltpu.VMEM_SHARED((N, D), jnp.float32),
        pltpu.SemaphoreType.DMA,
        pltpu.SemaphoreType.DMA,
    ],
)

_RB = 1000


def _mm_body(x_ref, w_ref, o_ref):
    o_ref[...] = jnp.dot(x_ref[...], w_ref[...],
                         preferred_element_type=jnp.float32)


def _mm(x, w):
    return pl.pallas_call(
        _mm_body,
        grid=(N // _RB,),
        in_specs=[pl.BlockSpec((_RB, D), lambda i: (i, 0)),
                  pl.BlockSpec((D, D), lambda i: (0, 0))],
        out_specs=pl.BlockSpec((_RB, D), lambda i: (i, 0)),
        out_shape=jax.ShapeDtypeStruct((N, D), jnp.float32),
    )(x, w)


def _normalize(p0, p1, d0, d1, f0):
    i = pl.program_id(0)
    riota = lax.broadcasted_iota(jnp.int32, (_RB, 1), 0)
    m = jnp.where((riota == 0) & (i == 0), PADTOT, 0.0)
    deg = d0[:, 0:1] + d1[:, 0:1] - m
    rinv = jnp.where(deg > 0, 1.0 / deg, 0.0)
    s = p0[...] + p1[...] - m * f0[...]
    return s, rinv


def _layer2_body(p0, p1, d0, d1, f0, b1, w2, o):
    s, rinv = _normalize(p0, p1, d0, d1, f0)
    h = s * rinv + b1[...]
    h = jnp.where(h >= 0, h, 0.2 * h)
    o[...] = jnp.dot(h, w2[...], preferred_element_type=jnp.float32)


def _final_body(p0, p1, d0, d1, f0, b2, o):
    s, rinv = _normalize(p0, p1, d0, d1, f0)
    h = s * rinv + b2[...]
    nrm = jnp.sqrt(jnp.sum(h * h, axis=1, keepdims=True))
    o[...] = h / jnp.maximum(nrm, 1e-12)


def _fused(body, extra_specs, p0, p1, d0, d1, f0, *rest):
    return pl.pallas_call(
        body,
        grid=(N // _RB,),
        in_specs=[pl.BlockSpec((_RB, D), lambda i: (i, 0)),
                  pl.BlockSpec((_RB, D), lambda i: (i, 0)),
                  pl.BlockSpec((_RB, DEGW), lambda i: (i, 0)),
                  pl.BlockSpec((_RB, DEGW), lambda i: (i, 0)),
                  pl.BlockSpec((1, D), lambda i: (0, 0))] + extra_specs,
        out_specs=pl.BlockSpec((_RB, D), lambda i: (i, 0)),
        out_shape=jax.ShapeDtypeStruct((N, D), jnp.float32),
    )(p0, p1, d0, d1, f0, *rest)


def kernel(x, edge_index, W1, b1, W2, b2):
    cols = edge_index[:, 0]
    rows = edge_index[:, 1]
    pad = E_PAD - E
    cols = jnp.concatenate([cols, jnp.zeros((pad,), jnp.int32)])
    rows = jnp.concatenate([rows, jnp.zeros((pad,), jnp.int32)])
    colsr = cols.reshape(NC, NS, NCH, CHUNK)
    rowsr = rows.reshape(NC, NS, NCH, CHUNK)
    zf = jnp.zeros((STRIPE, D), jnp.float32)
    zd = jnp.zeros((STRIPE, DEGW), jnp.float32)
    vec_spec = [pl.BlockSpec((1, D), lambda i: (0, 0))]
    mat_spec = vec_spec + [pl.BlockSpec((D, D), lambda i: (0, 0))]

    d0, d1 = _deg(rowsr, zd)
    y1 = _mm(x, W1)
    p0, p1 = _spmm(y1, colsr, rowsr, zf)
    y2 = _fused(_layer2_body, mat_spec, p0, p1, d0, d1, y1[0:1, :],
                b1.reshape(1, D), W2)
    q0, q1 = _spmm(y2, colsr, rowsr, zf)
    return _fused(_final_body, vec_spec, q0, q1, d0, d1, y2[0:1, :],
                  b2.reshape(1, D))


# speedup vs baseline: 15.5506x; 2.9693x over previous
import jax
import jax.numpy as jnp
from jax import lax
from jax.experimental import pallas as pl
from jax.experimental.pallas import tpu as pltpu
from jax.experimental.pallas import tpu_sc as plsc

N = 10000
E = 320000
D = 128
NC = 2
NS = 16
CHUNK = 128
NCH = 80
IB = 16
NGRP = NCH // IB
EPT = NCH * CHUNK
E_PAD = NC * NS * EPT
STRIPE = 632
LAST = N - 15 * STRIPE
DEGW = 128

_MESH = plsc.VectorSubcoreMesh(core_axis_name="c", subcore_axis_name="s",
                               num_cores=NC, num_subcores=NS)


def _stripe_copy(src, dst, sid):
    @pl.when(sid < 15)
    def _():
        pltpu.sync_copy(src.at[pl.ds(sid * STRIPE, STRIPE)],
                        dst.at[pl.ds(sid * STRIPE, STRIPE)])

    @pl.when(sid == 15)
    def _():
        pltpu.sync_copy(src.at[pl.ds(15 * STRIPE, LAST)],
                        dst.at[pl.ds(15 * STRIPE, LAST)])


def _zero_stripe(z, dst, sid):
    @pl.when(sid < 15)
    def _():
        pltpu.sync_copy(z, dst.at[pl.ds(sid * STRIPE, STRIPE)])

    @pl.when(sid == 15)
    def _():
        pltpu.sync_copy(z.at[pl.ds(0, LAST)], dst.at[pl.ds(15 * STRIPE, LAST)])


def _chunks_for(cid, sid):
    w = cid * NS + sid
    return jnp.clip((E - w * EPT) // CHUNK, 0, NCH)


def _deg_body(rowsr, zd, d0, d1, rows_v, ones_v, dacc):
    cid = lax.axis_index("c")
    sid = lax.axis_index("s")

    pltpu.sync_copy(rowsr.at[cid, sid], rows_v)
    _zero_stripe(zd, dacc, sid)
    one = jnp.ones((16,), jnp.float32)
    for i in range(CHUNK):
        for l in range(DEGW // 16):
            ones_v[i, pl.ds(l * 16, 16)] = one
    plsc.subcore_barrier()

    def step(j, carry):
        pltpu.sync_copy(ones_v, dacc.at[rows_v.at[j]], add=True)
        return carry

    lax.fori_loop(0, _chunks_for(cid, sid), step, 0)
    plsc.subcore_barrier()

    @pl.when(cid == 0)
    def _():
        _stripe_copy(dacc, d0, sid)

    @pl.when(cid == 1)
    def _():
        _stripe_copy(dacc, d1, sid)


_deg = pl.kernel(
    _deg_body,
    out_type=(jax.ShapeDtypeStruct((N, DEGW), jnp.float32),) * 2,
    mesh=_MESH,
    scratch_types=[
        pltpu.VMEM((NCH, CHUNK), jnp.int32),
        pltpu.VMEM((CHUNK, DEGW), jnp.float32),
        pltpu.VMEM_SHARED((N, DEGW), jnp.float32),
    ],
)


def _spmm_body(feat, colsr, rowsr, zf, p0, p1,
               cols_v, rows_v, gb0, gb1, acc, sem0, sem1):
    cid = lax.axis_index("c")
    sid = lax.axis_index("s")

    _zero_stripe(zf, acc, sid)
    nch = _chunks_for(cid, sid)
    plsc.subcore_barrier()

    def group(g, carry):
        nb = jnp.minimum(IB, nch - g * IB)
        pltpu.sync_copy(colsr.at[cid, sid, pl.ds(g * IB, IB)], cols_v)
        pltpu.sync_copy(rowsr.at[cid, sid, pl.ds(g * IB, IB)], rows_v)
        pltpu.async_copy(feat.at[cols_v.at[0]], gb0, sem0)

        def step(k, c2):
            for b in range(2):
                gbc, semc = (gb0, sem0) if b == 0 else (gb1, sem1)
                gbn, semn = (gb1, sem1) if b == 0 else (gb0, sem0)
                kk = 2 * k + b
                nxt = kk + 1

                @pl.when(nxt < nb)
                def _():
                    pltpu.async_copy(feat.at[cols_v.at[nxt]], gbn, semn)

                pltpu.make_async_copy(feat.at[cols_v.at[kk]], gbc, semc).wait()
                pltpu.sync_copy(gbc, acc.at[rows_v.at[kk]], add=True)
            return c2

        lax.fori_loop(0, nb // 2, step, 0)
        return carry

    lax.fori_loop(0, (nch + IB - 1) // IB, group, 0)

    plsc.subcore_barrier()

    @pl.when(cid == 0)
    def _():
        _stripe_copy(acc, p0, sid)

    @pl.when(cid == 1)
    def _():
        _stripe_copy(acc, p1, sid)


_spmm = pl.kernel(
    _spmm_body,
    out_type=(jax.ShapeDtypeStruct((N, D), jnp.float32),) * 2,
    mesh=_MESH,
    scratch_types=[
        pltpu.VMEM((IB, CHUNK), jnp.int32),
        pltpu.VMEM((IB, CHUNK), jnp.int32),
        pltpu.VMEM((CHUNK, D), jnp.float32),
        pltpu.VMEM((CHUNK, D), jnp.float32),
        pltpu.VMEM_SHARED((N, D), jnp.float32),
        pltpu.SemaphoreType.DMA,
        pltpu.SemaphoreType.DMA,
    ],
)

_RB = 1000


def _mm_body(x_ref, w_ref, o_ref):
    o_ref[...] = jnp.dot(x_ref[...], w_ref[...],
                         preferred_element_type=jnp.float32)


def _mm(x, w):
    return pl.pallas_call(
        _mm_body,
        grid=(N // _RB,),
        in_specs=[pl.BlockSpec((_RB, D), lambda i: (i, 0)),
                  pl.BlockSpec((D, D), lambda i: (0, 0))],
        out_specs=pl.BlockSpec((_RB, D), lambda i: (i, 0)),
        out_shape=jax.ShapeDtypeStruct((N, D), jnp.float32),
    )(x, w)


def _normalize(p0, p1, d0, d1):
    deg = d0[:, 0:1] + d1[:, 0:1]
    rinv = jnp.where(deg > 0, 1.0 / deg, 0.0)
    return p0[...] + p1[...], rinv


def _layer2_body(p0, p1, d0, d1, b1, w2, o):
    s, rinv = _normalize(p0, p1, d0, d1)
    h = s * rinv + b1[...]
    h = jnp.where(h >= 0, h, 0.2 * h)
    o[...] = jnp.dot(h, w2[...], preferred_element_type=jnp.float32)


def _final_body(p0, p1, d0, d1, b2, o):
    s, rinv = _normalize(p0, p1, d0, d1)
    h = s * rinv + b2[...]
    nrm = jnp.sqrt(jnp.sum(h * h, axis=1, keepdims=True))
    o[...] = h / jnp.maximum(nrm, 1e-12)


def _fused(body, extra_specs, p0, p1, d0, d1, *rest):
    return pl.pallas_call(
        body,
        grid=(N // _RB,),
        in_specs=[pl.BlockSpec((_RB, D), lambda i: (i, 0)),
                  pl.BlockSpec((_RB, D), lambda i: (i, 0)),
                  pl.BlockSpec((_RB, DEGW), lambda i: (i, 0)),
                  pl.BlockSpec((_RB, DEGW), lambda i: (i, 0))] + extra_specs,
        out_specs=pl.BlockSpec((_RB, D), lambda i: (i, 0)),
        out_shape=jax.ShapeDtypeStruct((N, D), jnp.float32),
    )(p0, p1, d0, d1, *rest)


def kernel(x, edge_index, W1, b1, W2, b2):
    cols = edge_index[:, 0]
    rows = edge_index[:, 1]
    pad = E_PAD - E
    cols = jnp.concatenate([cols, jnp.zeros((pad,), jnp.int32)])
    rows = jnp.concatenate([rows, jnp.zeros((pad,), jnp.int32)])
    colsr = cols.reshape(NC, NS, NCH, CHUNK)
    rowsr = rows.reshape(NC, NS, NCH, CHUNK)
    zf = jnp.zeros((STRIPE, D), jnp.float32)
    zd = jnp.zeros((STRIPE, DEGW), jnp.float32)
    vec_spec = [pl.BlockSpec((1, D), lambda i: (0, 0))]
    mat_spec = vec_spec + [pl.BlockSpec((D, D), lambda i: (0, 0))]

    d0, d1 = _deg(rowsr, zd)
    y1 = _mm(x, W1)
    p0, p1 = _spmm(y1, colsr, rowsr, zf)
    y2 = _fused(_layer2_body, mat_spec, p0, p1, d0, d1,
                b1.reshape(1, D), W2)
    q0, q1 = _spmm(y2, colsr, rowsr, zf)
    return _fused(_final_body, vec_spec, q0, q1, d0, d1,
                  b2.reshape(1, D))


# speedup vs baseline: 15.5842x; 1.0022x over previous
import jax
import jax.numpy as jnp
from jax import lax
from jax.experimental import pallas as pl
from jax.experimental.pallas import tpu as pltpu
from jax.experimental.pallas import tpu_sc as plsc

N = 10000
E = 320000
D = 128
NC = 2
NS = 16
CHUNK = 128
NCH = 80
IB = 16
NGRP = NCH // IB
EPT = NCH * CHUNK
E_PAD = NC * NS * EPT
STRIPE = 632
LAST = N - 15 * STRIPE
DEGW = 128

_MESH = plsc.VectorSubcoreMesh(core_axis_name="c", subcore_axis_name="s",
                               num_cores=NC, num_subcores=NS)


def _stripe_copy(src, dst, sid):
    @pl.when(sid < 15)
    def _():
        pltpu.sync_copy(src.at[pl.ds(sid * STRIPE, STRIPE)],
                        dst.at[pl.ds(sid * STRIPE, STRIPE)])

    @pl.when(sid == 15)
    def _():
        pltpu.sync_copy(src.at[pl.ds(15 * STRIPE, LAST)],
                        dst.at[pl.ds(15 * STRIPE, LAST)])


def _zero_stripe(z, dst, sid):
    @pl.when(sid < 15)
    def _():
        pltpu.sync_copy(z, dst.at[pl.ds(sid * STRIPE, STRIPE)])

    @pl.when(sid == 15)
    def _():
        pltpu.sync_copy(z.at[pl.ds(0, LAST)], dst.at[pl.ds(15 * STRIPE, LAST)])


def _chunks_for(cid, sid):
    w = cid * NS + sid
    return jnp.clip((E - w * EPT) // CHUNK, 0, NCH)


def _deg_body(rowsr, zd, d0, d1, rows_v, ones_v, dacc):
    cid = lax.axis_index("c")
    sid = lax.axis_index("s")

    pltpu.sync_copy(rowsr.at[cid, sid], rows_v)
    _zero_stripe(zd, dacc, sid)
    one = jnp.ones((16,), jnp.float32)
    for i in range(CHUNK):
        for l in range(DEGW // 16):
            ones_v[i, pl.ds(l * 16, 16)] = one
    plsc.subcore_barrier()

    def step(j, carry):
        pltpu.sync_copy(ones_v, dacc.at[rows_v.at[j]], add=True)
        return carry

    lax.fori_loop(0, _chunks_for(cid, sid), step, 0)
    plsc.subcore_barrier()

    @pl.when(cid == 0)
    def _():
        _stripe_copy(dacc, d0, sid)

    @pl.when(cid == 1)
    def _():
        _stripe_copy(dacc, d1, sid)


_deg = pl.kernel(
    _deg_body,
    out_type=(jax.ShapeDtypeStruct((N, DEGW), jnp.float32),) * 2,
    mesh=_MESH,
    scratch_types=[
        pltpu.VMEM((NCH, CHUNK), jnp.int32),
        pltpu.VMEM((CHUNK, DEGW), jnp.float32),
        pltpu.VMEM_SHARED((N, DEGW), jnp.float32),
    ],
)


def _spmm_body(feat, colsr, rowsr, zf, p0, p1,
               cols_v, rows_v, gb0, gb1, acc, sem0, sem1):
    cid = lax.axis_index("c")
    sid = lax.axis_index("s")

    _zero_stripe(zf, acc, sid)
    nch = _chunks_for(cid, sid)
    plsc.subcore_barrier()

    def group(g, carry):
        nb = jnp.minimum(IB, nch - g * IB)
        pltpu.sync_copy(colsr.at[cid, sid, pl.ds(g * IB, IB)], cols_v)
        pltpu.sync_copy(rowsr.at[cid, sid, pl.ds(g * IB, IB)], rows_v)
        pltpu.async_copy(feat.at[cols_v.at[0]], gb0, sem0)

        def step(k, c2):
            for b in range(2):
                gbc, semc = (gb0, sem0) if b == 0 else (gb1, sem1)
                gbn, semn = (gb1, sem1) if b == 0 else (gb0, sem0)
                kk = 2 * k + b
                nxt = kk + 1

                @pl.when(nxt < nb)
                def _():
                    pltpu.async_copy(feat.at[cols_v.at[nxt]], gbn, semn)

                pltpu.make_async_copy(feat.at[cols_v.at[kk]], gbc, semc).wait()
                pltpu.sync_copy(gbc, acc.at[rows_v.at[kk]], add=True)
            return c2

        lax.fori_loop(0, nb // 2, step, 0)
        return carry

    lax.fori_loop(0, (nch + IB - 1) // IB, group, 0)

    plsc.subcore_barrier()

    @pl.when(cid == 0)
    def _():
        _stripe_copy(acc, p0, sid)

    @pl.when(cid == 1)
    def _():
        _stripe_copy(acc, p1, sid)


_spmm = pl.kernel(
    _spmm_body,
    out_type=(jax.ShapeDtypeStruct((N, D), jnp.float32),) * 2,
    mesh=_MESH,
    scratch_types=[
        pltpu.VMEM((IB, CHUNK), jnp.int32),
        pltpu.VMEM((IB, CHUNK), jnp.int32),
        pltpu.VMEM((CHUNK, D), jnp.float32),
        pltpu.VMEM((CHUNK, D), jnp.float32),
        pltpu.VMEM_SHARED((N, D), jnp.float32),
        pltpu.SemaphoreType.DMA,
        pltpu.SemaphoreType.DMA,
    ],
)

_RB = 1000


def _mm_body(x_ref, w_ref, o_ref):
    o_ref[...] = jnp.dot(x_ref[...], w_ref[...],
                         preferred_element_type=jnp.float32)


def _mm(x, w):
    return pl.pallas_call(
        _mm_body,
        grid=(N // _RB,),
        in_specs=[pl.BlockSpec((_RB, D), lambda i: (i, 0)),
                  pl.BlockSpec((D, D), lambda i: (0, 0))],
        out_specs=pl.BlockSpec((_RB, D), lambda i: (i, 0)),
        out_shape=jax.ShapeDtypeStruct((N, D), jnp.float32),
    )(x, w)


def _normalize(p0, p1, d0, d1):
    deg = d0[:, 0:1] + d1[:, 0:1]
    rinv = jnp.where(deg > 0, 1.0 / deg, 0.0)
    return p0[...] + p1[...], rinv


def _layer2_body(p0, p1, d0, d1, b1, w2, o):
    s, rinv = _normalize(p0, p1, d0, d1)
    h = s * rinv + b1[...]
    h = jnp.where(h >= 0, h, 0.2 * h)
    o[...] = jnp.dot(h, w2[...], preferred_element_type=jnp.float32)


def _final_body(p0, p1, d0, d1, b2, o):
    s, rinv = _normalize(p0, p1, d0, d1)
    h = s * rinv + b2[...]
    nrm = jnp.sqrt(jnp.sum(h * h, axis=1, keepdims=True))
    o[...] = h / jnp.maximum(nrm, 1e-12)


def _fused(body, extra_specs, p0, p1, d0, d1, *rest):
    return pl.pallas_call(
        body,
        grid=(N // _RB,),
        in_specs=[pl.BlockSpec((_RB, D), lambda i: (i, 0)),
                  pl.BlockSpec((_RB, D), lambda i: (i, 0)),
                  pl.BlockSpec((_RB, 8), lambda i: (i, 0)),
                  pl.BlockSpec((_RB, 8), lambda i: (i, 0))] + extra_specs,
        out_specs=pl.BlockSpec((_RB, D), lambda i: (i, 0)),
        out_shape=jax.ShapeDtypeStruct((N, D), jnp.float32),
    )(p0, p1, d0, d1, *rest)


def kernel(x, edge_index, W1, b1, W2, b2):
    cols = edge_index[:, 0]
    rows = edge_index[:, 1]
    pad = E_PAD - E
    cols = jnp.concatenate([cols, jnp.zeros((pad,), jnp.int32)])
    rows = jnp.concatenate([rows, jnp.zeros((pad,), jnp.int32)])
    colsr = cols.reshape(NC, NS, NCH, CHUNK)
    rowsr = rows.reshape(NC, NS, NCH, CHUNK)
    zf = jnp.zeros((STRIPE, D), jnp.float32)
    zd = jnp.zeros((STRIPE, DEGW), jnp.float32)
    vec_spec = [pl.BlockSpec((1, D), lambda i: (0, 0))]
    mat_spec = vec_spec + [pl.BlockSpec((D, D), lambda i: (0, 0))]

    d0, d1 = _deg(rowsr, zd)
    d0s, d1s = d0[:, :8], d1[:, :8]
    y1 = _mm(x, W1)
    p0, p1 = _spmm(y1, colsr, rowsr, zf)
    y2 = _fused(_layer2_body, mat_spec, p0, p1, d0s, d1s,
                b1.reshape(1, D), W2)
    q0, q1 = _spmm(y2, colsr, rowsr, zf)
    return _fused(_final_body, vec_spec, q0, q1, d0s, d1s,
                  b2.reshape(1, D))


# speedup vs baseline: 16.5487x; 1.0619x over previous
import jax
import jax.numpy as jnp
from jax import lax
from jax.experimental import pallas as pl
from jax.experimental.pallas import tpu as pltpu
from jax.experimental.pallas import tpu_sc as plsc

N = 10000
E = 320000
D = 128
NC = 2
NS = 16
CHUNK = 100
NCH = 100
IB = 25
EPT = NCH * CHUNK
STRIPE = 632
LAST = N - 15 * STRIPE
DEGW = 128

_MESH = plsc.VectorSubcoreMesh(core_axis_name="c", subcore_axis_name="s",
                               num_cores=NC, num_subcores=NS)


def _stripe_copy(src, dst, sid):
    @pl.when(sid < 15)
    def _():
        pltpu.sync_copy(src.at[pl.ds(sid * STRIPE, STRIPE)],
                        dst.at[pl.ds(sid * STRIPE, STRIPE)])

    @pl.when(sid == 15)
    def _():
        pltpu.sync_copy(src.at[pl.ds(15 * STRIPE, LAST)],
                        dst.at[pl.ds(15 * STRIPE, LAST)])


def _zero_stripe(z, dst, sid):
    @pl.when(sid < 15)
    def _():
        pltpu.sync_copy(z, dst.at[pl.ds(sid * STRIPE, STRIPE)])

    @pl.when(sid == 15)
    def _():
        pltpu.sync_copy(z.at[pl.ds(0, LAST)], dst.at[pl.ds(15 * STRIPE, LAST)])


def _deg_body(rowsr, zd, d0, d1, rows_v, ones_v, dacc):
    cid = lax.axis_index("c")
    sid = lax.axis_index("s")

    for g in range(NCH // IB):
        pltpu.sync_copy(rowsr.at[cid, sid, g],
                        rows_v.at[pl.ds(g * IB, IB)])
    _zero_stripe(zd, dacc, sid)
    one = jnp.ones((16,), jnp.float32)
    for i in range(CHUNK):
        for l in range(DEGW // 16):
            ones_v[i, pl.ds(l * 16, 16)] = one
    plsc.subcore_barrier()

    def step(j, carry):
        pltpu.sync_copy(ones_v, dacc.at[rows_v.at[j]], add=True)
        return carry

    lax.fori_loop(0, NCH, step, 0)
    plsc.subcore_barrier()

    @pl.when(cid == 0)
    def _():
        _stripe_copy(dacc, d0, sid)

    @pl.when(cid == 1)
    def _():
        _stripe_copy(dacc, d1, sid)


_deg = pl.kernel(
    _deg_body,
    out_type=(jax.ShapeDtypeStruct((N, DEGW), jnp.float32),) * 2,
    mesh=_MESH,
    scratch_types=[
        pltpu.VMEM((NCH, CHUNK), jnp.int32),
        pltpu.VMEM((CHUNK, DEGW), jnp.float32),
        pltpu.VMEM_SHARED((N, DEGW), jnp.float32),
    ],
)


def _spmm_body(feat, colsr, rowsr, zf, p0, p1,
               cols_v, rows_v, gb, acc, gsem, ssem):
    cid = lax.axis_index("c")
    sid = lax.axis_index("s")

    _zero_stripe(zf, acc, sid)
    plsc.subcore_barrier()

    def group(g, carry):
        pltpu.sync_copy(colsr.at[cid, sid, g], cols_v)
        pltpu.sync_copy(rowsr.at[cid, sid, g], rows_v)
        pltpu.async_copy(feat.at[cols_v.at[0]], gb.at[0], gsem.at[0])
        pltpu.async_copy(feat.at[cols_v.at[1]], gb.at[1], gsem.at[1])

        def step(j, c2):
            b = lax.rem(j, 3)
            bn = lax.rem(j + 2, 3)
            pltpu.make_async_copy(feat.at[cols_v.at[j]],
                                  gb.at[b], gsem.at[b]).wait()
            pltpu.async_copy(gb.at[b], acc.at[rows_v.at[j]], ssem.at[b],
                             add=True)

            @pl.when(j >= 1)
            def _():
                pltpu.make_async_copy(gb.at[bn], acc.at[rows_v.at[j - 1]],
                                      ssem.at[bn]).wait()

            @pl.when(j + 2 < IB)
            def _():
                pltpu.async_copy(feat.at[cols_v.at[j + 2]],
                                 gb.at[bn], gsem.at[bn])
            return c2

        lax.fori_loop(0, IB, step, 0)
        pltpu.make_async_copy(gb.at[(IB - 1) % 3],
                              acc.at[rows_v.at[IB - 1]],
                              ssem.at[(IB - 1) % 3]).wait()
        return carry

    lax.fori_loop(0, NCH // IB, group, 0)

    plsc.subcore_barrier()

    @pl.when(cid == 0)
    def _():
        _stripe_copy(acc, p0, sid)

    @pl.when(cid == 1)
    def _():
        _stripe_copy(acc, p1, sid)


_spmm = pl.kernel(
    _spmm_body,
    out_type=(jax.ShapeDtypeStruct((N, D), jnp.float32),) * 2,
    mesh=_MESH,
    scratch_types=[
        pltpu.VMEM((IB, CHUNK), jnp.int32),
        pltpu.VMEM((IB, CHUNK), jnp.int32),
        pltpu.VMEM((3, CHUNK, D), jnp.float32),
        pltpu.VMEM_SHARED((N, D), jnp.float32),
        pltpu.SemaphoreType.DMA((3,)),
        pltpu.SemaphoreType.DMA((3,)),
    ],
)

_RB = 1000


def _mm_body(x_ref, w_ref, o_ref):
    o_ref[...] = jnp.dot(x_ref[...], w_ref[...],
                         preferred_element_type=jnp.float32)


def _mm(x, w):
    return pl.pallas_call(
        _mm_body,
        grid=(N // _RB,),
        in_specs=[pl.BlockSpec((_RB, D), lambda i: (i, 0)),
                  pl.BlockSpec((D, D), lambda i: (0, 0))],
        out_specs=pl.BlockSpec((_RB, D), lambda i: (i, 0)),
        out_shape=jax.ShapeDtypeStruct((N, D), jnp.float32),
    )(x, w)


def _normalize(p0, p1, d0, d1):
    deg = d0[:, 0:1] + d1[:, 0:1]
    rinv = jnp.where(deg > 0, 1.0 / deg, 0.0)
    return p0[...] + p1[...], rinv


def _layer2_body(p0, p1, d0, d1, b1, w2, o):
    s, rinv = _normalize(p0, p1, d0, d1)
    h = s * rinv + b1[...]
    h = jnp.where(h >= 0, h, 0.2 * h)
    o[...] = jnp.dot(h, w2[...], preferred_element_type=jnp.float32)


def _final_body(p0, p1, d0, d1, b2, o):
    s, rinv = _normalize(p0, p1, d0, d1)
    h = s * rinv + b2[...]
    nrm = jnp.sqrt(jnp.sum(h * h, axis=1, keepdims=True))
    o[...] = h / jnp.maximum(nrm, 1e-12)


def _fused(body, extra_specs, p0, p1, d0, d1, *rest):
    return pl.pallas_call(
        body,
        grid=(N // _RB,),
        in_specs=[pl.BlockSpec((_RB, D), lambda i: (i, 0)),
                  pl.BlockSpec((_RB, D), lambda i: (i, 0)),
                  pl.BlockSpec((_RB, 8), lambda i: (i, 0)),
                  pl.BlockSpec((_RB, 8), lambda i: (i, 0))] + extra_specs,
        out_specs=pl.BlockSpec((_RB, D), lambda i: (i, 0)),
        out_shape=jax.ShapeDtypeStruct((N, D), jnp.float32),
    )(p0, p1, d0, d1, *rest)


def kernel(x, edge_index, W1, b1, W2, b2):
    cols = edge_index[:, 0]
    rows = edge_index[:, 1]
    colsr = cols.reshape(NC, NS, NCH // IB, IB, CHUNK)
    rowsr = rows.reshape(NC, NS, NCH // IB, IB, CHUNK)
    zf = jnp.zeros((STRIPE, D), jnp.float32)
    zd = jnp.zeros((STRIPE, DEGW), jnp.float32)
    vec_spec = [pl.BlockSpec((1, D), lambda i: (0, 0))]
    mat_spec = vec_spec + [pl.BlockSpec((D, D), lambda i: (0, 0))]

    d0, d1 = _deg(rowsr, zd)
    d0s, d1s = d0[:, :8], d1[:, :8]
    y1 = _mm(x, W1)
    p0, p1 = _spmm(y1, colsr, rowsr, zf)
    y2 = _fused(_layer2_body, mat_spec, p0, p1, d0s, d1s,
                b1.reshape(1, D), W2)
    q0, q1 = _spmm(y2, colsr, rowsr, zf)
    return _fused(_final_body, vec_spec, q0, q1, d0s, d1s,
                  b2.reshape(1, D))


# speedup vs baseline: 19.7057x; 1.1908x over previous
import jax
import jax.numpy as jnp
from jax import lax
from jax.experimental import pallas as pl
from jax.experimental.pallas import tpu as pltpu
from jax.experimental.pallas import tpu_sc as plsc

N = 10000
E = 320000
D = 128
NC = 2
NS = 16
CHUNK = 100
NCH = 100
IB = 25
EPT = NCH * CHUNK
STRIPE = 632
LAST = N - 15 * STRIPE
NP = 10240
SEG = NP // NS

_MESH = plsc.VectorSubcoreMesh(core_axis_name="c", subcore_axis_name="s",
                               num_cores=NC, num_subcores=NS)


def _stripe_copy(src, dst, sid):
    @pl.when(sid < 15)
    def _():
        pltpu.sync_copy(src.at[pl.ds(sid * STRIPE, STRIPE)],
                        dst.at[pl.ds(sid * STRIPE, STRIPE)])

    @pl.when(sid == 15)
    def _():
        pltpu.sync_copy(src.at[pl.ds(15 * STRIPE, LAST)],
                        dst.at[pl.ds(15 * STRIPE, LAST)])


def _zero_stripe(z, dst, sid):
    @pl.when(sid < 15)
    def _():
        pltpu.sync_copy(z, dst.at[pl.ds(sid * STRIPE, STRIPE)])

    @pl.when(sid == 15)
    def _():
        pltpu.sync_copy(z.at[pl.ds(0, LAST)], dst.at[pl.ds(15 * STRIPE, LAST)])


def _deg_body(rows2, d0, d1, rows_v, hist, rbuf, dseg, shp):
    cid = lax.axis_index("c")
    sid = lax.axis_index("s")

    pltpu.sync_copy(rows2.at[cid, sid], rows_v)
    z = jnp.zeros((16,), jnp.float32)

    def zstep(i, carry):
        hist[pl.ds(i * 16, 16)] = z
        return carry

    lax.fori_loop(0, NP // 16, zstep, 0)
    ones = jnp.ones((16,), jnp.float32)

    def step(i, carry):
        plsc.addupdate_scatter(hist, [rows_v[i]], ones)
        return carry

    lax.fori_loop(0, EPT // 16, step, 0)
    pltpu.sync_copy(hist, shp.at[sid])
    plsc.subcore_barrier()
    for t in range(NS):
        pltpu.sync_copy(shp.at[t, pl.ds(sid * SEG, SEG)], rbuf.at[t])

    def rstep(c, carry):
        acc = rbuf[0, pl.ds(c * 16, 16)]
        for t in range(1, NS):
            acc = acc + rbuf[t, pl.ds(c * 16, 16)]
        dseg[pl.ds(c * 16, 16)] = acc
        return carry

    lax.fori_loop(0, SEG // 16, rstep, 0)

    @pl.when(cid == 0)
    def _():
        pltpu.sync_copy(dseg, d0.at[pl.ds(sid * SEG, SEG)])

    @pl.when(cid == 1)
    def _():
        pltpu.sync_copy(dseg, d1.at[pl.ds(sid * SEG, SEG)])


_deg = pl.kernel(
    _deg_body,
    out_type=(jax.ShapeDtypeStruct((NP,), jnp.float32),) * 2,
    mesh=_MESH,
    scratch_types=[
        pltpu.VMEM((EPT // 16, 16), jnp.int32),
        pltpu.VMEM((NP,), jnp.float32),
        pltpu.VMEM((NS, SEG), jnp.float32),
        pltpu.VMEM((SEG,), jnp.float32),
        pltpu.VMEM_SHARED((NS, NP), jnp.float32),
    ],
    compiler_params=pltpu.CompilerParams(needs_layout_passes=False),
)


def _spmm_body(feat, colsr, rowsr, zf, p0, p1,
               cols_v, rows_v, gb, acc, gsem, ssem):
    cid = lax.axis_index("c")
    sid = lax.axis_index("s")

    _zero_stripe(zf, acc, sid)
    plsc.subcore_barrier()

    def group(g, carry):
        pltpu.sync_copy(colsr.at[cid, sid, g], cols_v)
        pltpu.sync_copy(rowsr.at[cid, sid, g], rows_v)
        pltpu.async_copy(feat.at[cols_v.at[0]], gb.at[0], gsem.at[0])
        pltpu.async_copy(feat.at[cols_v.at[1]], gb.at[1], gsem.at[1])

        def step(j, c2):
            b = lax.rem(j, 3)
            bn = lax.rem(j + 2, 3)
            pltpu.make_async_copy(feat.at[cols_v.at[j]],
                                  gb.at[b], gsem.at[b]).wait()
            pltpu.async_copy(gb.at[b], acc.at[rows_v.at[j]], ssem.at[b],
                             add=True)

            @pl.when(j >= 1)
            def _():
                pltpu.make_async_copy(gb.at[bn], acc.at[rows_v.at[j - 1]],
                                      ssem.at[bn]).wait()

            @pl.when(j + 2 < IB)
            def _():
                pltpu.async_copy(feat.at[cols_v.at[j + 2]],
                                 gb.at[bn], gsem.at[bn])
            return c2

        lax.fori_loop(0, IB, step, 0)
        pltpu.make_async_copy(gb.at[(IB - 1) % 3],
                              acc.at[rows_v.at[IB - 1]],
                              ssem.at[(IB - 1) % 3]).wait()
        return carry

    lax.fori_loop(0, NCH // IB, group, 0)

    plsc.subcore_barrier()

    @pl.when(cid == 0)
    def _():
        _stripe_copy(acc, p0, sid)

    @pl.when(cid == 1)
    def _():
        _stripe_copy(acc, p1, sid)


_spmm = pl.kernel(
    _spmm_body,
    out_type=(jax.ShapeDtypeStruct((N, D), jnp.float32),) * 2,
    mesh=_MESH,
    scratch_types=[
        pltpu.VMEM((IB, CHUNK), jnp.int32),
        pltpu.VMEM((IB, CHUNK), jnp.int32),
        pltpu.VMEM((3, CHUNK, D), jnp.float32),
        pltpu.VMEM_SHARED((N, D), jnp.float32),
        pltpu.SemaphoreType.DMA((3,)),
        pltpu.SemaphoreType.DMA((3,)),
    ],
)

_RB = 1000


def _mm_body(x_ref, w_ref, o_ref):
    o_ref[...] = jnp.dot(x_ref[...], w_ref[...],
                         preferred_element_type=jnp.float32)


def _mm(x, w):
    return pl.pallas_call(
        _mm_body,
        grid=(N // _RB,),
        in_specs=[pl.BlockSpec((_RB, D), lambda i: (i, 0)),
                  pl.BlockSpec((D, D), lambda i: (0, 0))],
        out_specs=pl.BlockSpec((_RB, D), lambda i: (i, 0)),
        out_shape=jax.ShapeDtypeStruct((N, D), jnp.float32),
    )(x, w)


def _normalize(p0, p1, d0, d1):
    deg = d0[...] + d1[...]
    rinv = jnp.where(deg > 0, 1.0 / deg, 0.0)
    return p0[...] + p1[...], rinv


def _layer2_body(p0, p1, d0, d1, b1, w2, o):
    s, rinv = _normalize(p0, p1, d0, d1)
    h = s * rinv + b1[...]
    h = jnp.where(h >= 0, h, 0.2 * h)
    o[...] = jnp.dot(h, w2[...], preferred_element_type=jnp.float32)


def _final_body(p0, p1, d0, d1, b2, o):
    s, rinv = _normalize(p0, p1, d0, d1)
    h = s * rinv + b2[...]
    nrm = jnp.sqrt(jnp.sum(h * h, axis=1, keepdims=True))
    o[...] = h / jnp.maximum(nrm, 1e-12)


def _fused(body, extra_specs, p0, p1, d0, d1, *rest):
    return pl.pallas_call(
        body,
        grid=(N // _RB,),
        in_specs=[pl.BlockSpec((_RB, D), lambda i: (i, 0)),
                  pl.BlockSpec((_RB, D), lambda i: (i, 0)),
                  pl.BlockSpec((_RB, 1), lambda i: (i, 0)),
                  pl.BlockSpec((_RB, 1), lambda i: (i, 0))] + extra_specs,
        out_specs=pl.BlockSpec((_RB, D), lambda i: (i, 0)),
        out_shape=jax.ShapeDtypeStruct((N, D), jnp.float32),
    )(p0, p1, d0, d1, *rest)


def kernel(x, edge_index, W1, b1, W2, b2):
    cols = edge_index[:, 0]
    rows = edge_index[:, 1]
    colsr = cols.reshape(NC, NS, NCH // IB, IB, CHUNK)
    rowsr = rows.reshape(NC, NS, NCH // IB, IB, CHUNK)
    rows2 = rows.reshape(NC, NS, EPT // 16, 16)
    zf = jnp.zeros((STRIPE, D), jnp.float32)
    vec_spec = [pl.BlockSpec((1, D), lambda i: (0, 0))]
    mat_spec = vec_spec + [pl.BlockSpec((D, D), lambda i: (0, 0))]

    d0, d1 = _deg(rows2)
    d0s = d0[:N].reshape(N, 1)
    d1s = d1[:N].reshape(N, 1)
    y1 = _mm(x, W1)
    p0, p1 = _spmm(y1, colsr, rowsr, zf)
    y2 = _fused(_layer2_body, mat_spec, p0, p1, d0s, d1s,
                b1.reshape(1, D), W2)
    q0, q1 = _spmm(y2, colsr, rowsr, zf)
    return _fused(_final_body, vec_spec, q0, q1, d0s, d1s,
                  b2.reshape(1, D))


# speedup vs baseline: 20.8118x; 1.0561x over previous
import jax
import jax.numpy as jnp
from jax import lax
from jax.experimental import pallas as pl
from jax.experimental.pallas import tpu as pltpu
from jax.experimental.pallas import tpu_sc as plsc

N = 10000
E = 320000
D = 128
NC = 2
NS = 16
CHUNK = 100
NCH = 100
IB = 25
EPT = NCH * CHUNK
STRIPE = 632
LAST = N - 15 * STRIPE
NP = 10240
SEG = NP // NS

_MESH = plsc.VectorSubcoreMesh(core_axis_name="c", subcore_axis_name="s",
                               num_cores=NC, num_subcores=NS)


def _stripe_copy(src, dst, sid):
    @pl.when(sid < 15)
    def _():
        pltpu.sync_copy(src.at[pl.ds(sid * STRIPE, STRIPE)],
                        dst.at[pl.ds(sid * STRIPE, STRIPE)])

    @pl.when(sid == 15)
    def _():
        pltpu.sync_copy(src.at[pl.ds(15 * STRIPE, LAST)],
                        dst.at[pl.ds(15 * STRIPE, LAST)])


def _zero_stripe(z, dst, sid):
    @pl.when(sid < 15)
    def _():
        pltpu.sync_copy(z, dst.at[pl.ds(sid * STRIPE, STRIPE)])

    @pl.when(sid == 15)
    def _():
        pltpu.sync_copy(z.at[pl.ds(0, LAST)], dst.at[pl.ds(15 * STRIPE, LAST)])


def _deg_body(rows_flat, d0, d1, rows_v, hist, rbuf, dseg, shp):
    cid = lax.axis_index("c")
    sid = lax.axis_index("s")

    wid = cid * NS + sid
    pltpu.sync_copy(rows_flat.at[pl.ds(wid * EPT, EPT)], rows_v)
    z = jnp.zeros((16,), jnp.float32)

    def zstep(i, carry):
        hist[pl.ds(i * 16, 16)] = z
        return carry

    lax.fori_loop(0, NP // 16, zstep, 0)
    ones = jnp.ones((16,), jnp.float32)

    def step(i, carry):
        plsc.addupdate_scatter(hist, [rows_v[pl.ds(i * 16, 16)]], ones)
        return carry

    lax.fori_loop(0, EPT // 16, step, 0)
    pltpu.sync_copy(hist, shp.at[sid])
    plsc.subcore_barrier()
    for t in range(NS):
        pltpu.sync_copy(shp.at[t, pl.ds(sid * SEG, SEG)], rbuf.at[t])

    def rstep(c, carry):
        acc = rbuf[0, pl.ds(c * 16, 16)]
        for t in range(1, NS):
            acc = acc + rbuf[t, pl.ds(c * 16, 16)]
        dseg[pl.ds(c * 16, 16)] = acc
        return carry

    lax.fori_loop(0, SEG // 16, rstep, 0)

    @pl.when(cid == 0)
    def _():
        pltpu.sync_copy(dseg, d0.at[pl.ds(sid * SEG, SEG)])

    @pl.when(cid == 1)
    def _():
        pltpu.sync_copy(dseg, d1.at[pl.ds(sid * SEG, SEG)])


_deg = pl.kernel(
    _deg_body,
    out_type=(jax.ShapeDtypeStruct((NP,), jnp.float32),) * 2,
    mesh=_MESH,
    scratch_types=[
        pltpu.VMEM((EPT,), jnp.int32),
        pltpu.VMEM((NP,), jnp.float32),
        pltpu.VMEM((NS, SEG), jnp.float32),
        pltpu.VMEM((SEG,), jnp.float32),
        pltpu.VMEM_SHARED((NS, NP), jnp.float32),
    ],
    compiler_params=pltpu.CompilerParams(needs_layout_passes=False),
)


def _spmm_body(feat, colsr, rowsr, zf, p0, p1,
               cols_v, rows_v, gb, acc, gsem, ssem):
    cid = lax.axis_index("c")
    sid = lax.axis_index("s")

    _zero_stripe(zf, acc, sid)
    plsc.subcore_barrier()

    def group(g, carry):
        pltpu.sync_copy(colsr.at[cid, sid, g], cols_v)
        pltpu.sync_copy(rowsr.at[cid, sid, g], rows_v)
        pltpu.async_copy(feat.at[cols_v.at[0]], gb.at[0], gsem.at[0])
        pltpu.async_copy(feat.at[cols_v.at[1]], gb.at[1], gsem.at[1])

        def step(j, c2):
            b = lax.rem(j, 3)
            bn = lax.rem(j + 2, 3)
            pltpu.make_async_copy(feat.at[cols_v.at[j]],
                                  gb.at[b], gsem.at[b]).wait()
            pltpu.async_copy(gb.at[b], acc.at[rows_v.at[j]], ssem.at[b],
                             add=True)

            @pl.when(j >= 1)
            def _():
                pltpu.make_async_copy(gb.at[bn], acc.at[rows_v.at[j - 1]],
                                      ssem.at[bn]).wait()

            @pl.when(j + 2 < IB)
            def _():
                pltpu.async_copy(feat.at[cols_v.at[j + 2]],
                                 gb.at[bn], gsem.at[bn])
            return c2

        lax.fori_loop(0, IB, step, 0)
        pltpu.make_async_copy(gb.at[(IB - 1) % 3],
                              acc.at[rows_v.at[IB - 1]],
                              ssem.at[(IB - 1) % 3]).wait()
        return carry

    lax.fori_loop(0, NCH // IB, group, 0)

    plsc.subcore_barrier()

    @pl.when(cid == 0)
    def _():
        _stripe_copy(acc, p0, sid)

    @pl.when(cid == 1)
    def _():
        _stripe_copy(acc, p1, sid)


_spmm = pl.kernel(
    _spmm_body,
    out_type=(jax.ShapeDtypeStruct((N, D), jnp.float32),) * 2,
    mesh=_MESH,
    scratch_types=[
        pltpu.VMEM((IB, CHUNK), jnp.int32),
        pltpu.VMEM((IB, CHUNK), jnp.int32),
        pltpu.VMEM((3, CHUNK, D), jnp.float32),
        pltpu.VMEM_SHARED((N, D), jnp.float32),
        pltpu.SemaphoreType.DMA((3,)),
        pltpu.SemaphoreType.DMA((3,)),
    ],
)

_RB = 2000


def _mm_body(x_ref, w_ref, o_ref):
    o_ref[...] = jnp.dot(x_ref[...], w_ref[...],
                         preferred_element_type=jnp.float32)


def _mm(x, w):
    return pl.pallas_call(
        _mm_body,
        grid=(N // _RB,),
        in_specs=[pl.BlockSpec((_RB, D), lambda i: (i, 0)),
                  pl.BlockSpec((D, D), lambda i: (0, 0))],
        out_specs=pl.BlockSpec((_RB, D), lambda i: (i, 0)),
        out_shape=jax.ShapeDtypeStruct((N, D), jnp.float32),
    )(x, w)


def _normalize(p0, p1, d0, d1):
    deg = d0[...] + d1[...]
    rinv = jnp.where(deg > 0, 1.0 / deg, 0.0)
    return p0[...] + p1[...], rinv


def _layer2_body(p0, p1, d0, d1, b1, w2, o):
    s, rinv = _normalize(p0, p1, d0, d1)
    h = s * rinv + b1[...]
    h = jnp.where(h >= 0, h, 0.2 * h)
    o[...] = jnp.dot(h, w2[...], preferred_element_type=jnp.float32)


def _final_body(p0, p1, d0, d1, b2, o):
    s, rinv = _normalize(p0, p1, d0, d1)
    h = s * rinv + b2[...]
    nrm = jnp.sqrt(jnp.sum(h * h, axis=1, keepdims=True))
    o[...] = h / jnp.maximum(nrm, 1e-12)


def _fused(body, extra_specs, p0, p1, d0, d1, *rest):
    return pl.pallas_call(
        body,
        grid=(N // _RB,),
        in_specs=[pl.BlockSpec((_RB, D), lambda i: (i, 0)),
                  pl.BlockSpec((_RB, D), lambda i: (i, 0)),
                  pl.BlockSpec((_RB, 1), lambda i: (i, 0)),
                  pl.BlockSpec((_RB, 1), lambda i: (i, 0))] + extra_specs,
        out_specs=pl.BlockSpec((_RB, D), lambda i: (i, 0)),
        out_shape=jax.ShapeDtypeStruct((N, D), jnp.float32),
    )(p0, p1, d0, d1, *rest)


def kernel(x, edge_index, W1, b1, W2, b2):
    cols = edge_index[:, 0]
    rows = edge_index[:, 1]
    colsr = cols.reshape(NC, NS, NCH // IB, IB, CHUNK)
    rowsr = rows.reshape(NC, NS, NCH // IB, IB, CHUNK)
    zf = jnp.zeros((STRIPE, D), jnp.float32)
    vec_spec = [pl.BlockSpec((1, D), lambda i: (0, 0))]
    mat_spec = vec_spec + [pl.BlockSpec((D, D), lambda i: (0, 0))]

    d0, d1 = _deg(rows)
    d0s = d0[:N].reshape(N, 1)
    d1s = d1[:N].reshape(N, 1)
    y1 = _mm(x, W1)
    p0, p1 = _spmm(y1, colsr, rowsr, zf)
    y2 = _fused(_layer2_body, mat_spec, p0, p1, d0s, d1s,
                b1.reshape(1, D), W2)
    q0, q1 = _spmm(y2, colsr, rowsr, zf)
    return _fused(_final_body, vec_spec, q0, q1, d0s, d1s,
                  b2.reshape(1, D))
